# baseline (speedup 1.0000x reference)
import functools

import jax
import jax.numpy as jnp
from jax import lax
from jax.experimental import pallas as pl
from jax.experimental.pallas import tpu as pltpu
from jax.experimental.pallas import tpu_sc as plsc

EMBED_DIM = 1024
NUM_CORES = 2
NUM_SUBCORES = 16
NUM_WORKERS = NUM_CORES * NUM_SUBCORES
LANES = 16
DRAIN_ROWS = 64


@functools.partial(jax.jit, static_argnames=("rows", "vocab"))
def _sc_embedding_lookup(table, idx, *, rows, vocab):
    rows_per_worker = rows // NUM_WORKERS
    n_drains = rows_per_worker // DRAIN_ROWS
    mesh = plsc.VectorSubcoreMesh(
        core_axis_name="c", subcore_axis_name="s", num_cores=NUM_CORES
    )

    @functools.partial(
        pl.kernel,
        out_type=jax.ShapeDtypeStruct((rows, EMBED_DIM), jnp.float32),
        mesh=mesh,
        scratch_types=[
            pltpu.VMEM((rows_per_worker,), jnp.int32),
            pltpu.VMEM((vocab, EMBED_DIM), jnp.float32),
            pltpu.VMEM((DRAIN_ROWS, EMBED_DIM), jnp.float32),
            pltpu.SemaphoreType.DMA,
        ],
    )
    def body(table_hbm, idx_hbm, out_hbm, idx_v, tab_v, drain_v, sem):
        wid = lax.axis_index("s") * NUM_CORES + lax.axis_index("c")
        base = wid * rows_per_worker
        pltpu.sync_copy(table_hbm, tab_v)
        pltpu.sync_copy(idx_hbm.at[pl.ds(base, rows_per_worker)], idx_v)
        @pl.loop(0, rows_per_worker // LANES)
        def _fire(vg):
            vec = idx_v[pl.ds(vg * LANES, LANES)]
            for l in range(LANES):
                i = vec[l]
                pltpu.async_copy(
                    tab_v.at[pl.ds(i, 1)],
                    out_hbm.at[pl.ds(base + vg * LANES + l, 1)],
                    sem,
                )

        @pl.loop(0, n_drains)
        def _drain(_):
            pltpu.make_async_copy(
                out_hbm.at[pl.ds(base, DRAIN_ROWS)], drain_v, sem
            ).wait()

    return body(table, idx)


def kernel(x, table):
    b, s = x.shape
    rows = b * s
    idx = x.reshape(rows).astype(jnp.int32)
    out = _sc_embedding_lookup(table, idx, rows=rows, vocab=table.shape[0])
    return out.reshape(b, s, EMBED_DIM)
